# Initial kernel scaffold; baseline (speedup 1.0000x reference)
import jax
import jax.numpy as jnp
from jax.experimental import pallas as pl


def kernel(x, edge_index, W_in, b_in, W_imp1, b_imp1, W_imp2, b_imp2, W_c1, b_c1, W_c2, b_c2, W_out, b_out):
    raise NotImplementedError("write your pallas kernel here")


# speedup vs baseline: 19.7731x; 19.7731x over previous
import functools

import jax
import jax.numpy as jnp
from jax import lax
from jax.experimental import pallas as pl
from jax.experimental.pallas import tpu as pltpu
from jax.experimental.pallas import tpu_sc as plsc

N = 10000
E = 320000
H = 128
NPAD = 10240
NC = 2
NS = 16
NW = NC * NS
L = 16
EPT = E // NW
CAP = NPAD
KCH = 128
GR2 = 16
SEG = NPAD // NW
PCAP = 4096
BR = 256


def _a_body(x_ref, wi_ref, bi_ref, w1_ref, b1_ref, w2_ref, b2_ref,
            h_ref, imp_ref):
    h = jax.nn.relu(jnp.dot(x_ref[...], wi_ref[...],
                            preferred_element_type=jnp.float32) + bi_ref[...])
    t = jax.nn.relu(jnp.dot(h, w1_ref[...],
                            preferred_element_type=jnp.float32) + b1_ref[...])
    z = jnp.dot(t, w2_ref[...], preferred_element_type=jnp.float32) + b2_ref[...]
    h_ref[...] = h
    imp_ref[...] = jax.nn.sigmoid(z)


def _run_a(x_pad, W_in, b_in, W_imp1, b_imp1, W_imp2, b_imp2):
    grid = NPAD // BR
    return pl.pallas_call(
        _a_body,
        grid=(grid,),
        in_specs=[
            pl.BlockSpec((BR, H), lambda i: (i, 0)),
            pl.BlockSpec((H, H), lambda i: (0, 0)),
            pl.BlockSpec((1, H), lambda i: (0, 0)),
            pl.BlockSpec((H, H // 2), lambda i: (0, 0)),
            pl.BlockSpec((1, H // 2), lambda i: (0, 0)),
            pl.BlockSpec((H // 2, 1), lambda i: (0, 0)),
            pl.BlockSpec((1, 1), lambda i: (0, 0)),
        ],
        out_specs=[
            pl.BlockSpec((BR, H), lambda i: (i, 0)),
            pl.BlockSpec((BR, 1), lambda i: (i, 0)),
        ],
        out_shape=[
            jax.ShapeDtypeStruct((NPAD, H), jnp.float32),
            jax.ShapeDtypeStruct((NPAD, 1), jnp.float32),
        ],
    )(x_pad, W_in, b_in.reshape(1, H), W_imp1, b_imp1.reshape(1, H // 2),
      W_imp2, b_imp2.reshape(1, 1))


def _b1_body(imp_hbm, row_hbm, col_hbm, ew_hbm, part_hbm,
             imp_v, row_v, col_v, ew_v, part_v):
    c = lax.axis_index("c")
    s = lax.axis_index("s")
    wid = s * NC + c
    base = wid * EPT
    pltpu.sync_copy(imp_hbm, imp_v)
    pltpu.sync_copy(row_hbm.at[pl.ds(base, EPT)], row_v)
    pltpu.sync_copy(col_hbm.at[pl.ds(base, EPT)], col_v)

    def body(i, carry):
        sacc, qacc = carry
        r = row_v[pl.ds(i * L, L)]
        cc = col_v[pl.ds(i * L, L)]
        ir = plsc.load_gather(imp_v, [r])
        ic = plsc.load_gather(imp_v, [cc])
        ew = (ir + ic) * 0.5
        ew_v[pl.ds(i * L, L)] = ew
        return (sacc + ew, qacc + ew * ew)

    z16 = jnp.zeros((L,), jnp.float32)
    sacc, qacc = lax.fori_loop(0, EPT // L, body, (z16, z16))
    sv = jnp.sum(sacc)
    qv = jnp.sum(qacc)
    lane = lax.iota(jnp.int32, L)
    part_v[...] = jnp.where(lane == 0, sv, jnp.where(lane == 1, qv, 0.0))
    pltpu.sync_copy(ew_v, ew_hbm.at[pl.ds(base, EPT)])
    pltpu.sync_copy(part_v, part_hbm.at[wid])


def _run_b1(imp_flat, row, col):
    mesh = plsc.VectorSubcoreMesh(core_axis_name="c", subcore_axis_name="s")
    f = pl.kernel(
        _b1_body,
        out_type=[
            jax.ShapeDtypeStruct((E,), jnp.float32),
            jax.ShapeDtypeStruct((NW, L), jnp.float32),
        ],
        mesh=mesh,
        compiler_params=pltpu.CompilerParams(needs_layout_passes=False),
        scratch_types=[
            pltpu.VMEM((NPAD,), jnp.float32),
            pltpu.VMEM((EPT,), jnp.int32),
            pltpu.VMEM((EPT,), jnp.int32),
            pltpu.VMEM((EPT,), jnp.float32),
            pltpu.VMEM((L,), jnp.float32),
        ],
    )
    return f(imp_flat, row, col)


def _b2_body(ew_hbm, row_hbm, col_hbm, thr_hbm,
             rowc_hbm, colc_hbm, cnt_hbm, degp_hbm,
             ew_v, row_v, col_v, rowc_v, colc_v, deg_v, thr_v, cnt_v):
    c = lax.axis_index("c")
    s = lax.axis_index("s")
    wid = s * NC + c
    base = wid * EPT
    pltpu.sync_copy(ew_hbm.at[pl.ds(base, EPT)], ew_v)
    pltpu.sync_copy(row_hbm.at[pl.ds(base, EPT)], row_v)
    pltpu.sync_copy(col_hbm.at[pl.ds(base, EPT)], col_v)
    pltpu.sync_copy(thr_hbm, thr_v)

    def zbody(i, _):
        deg_v[pl.ds(i * L, L)] = jnp.zeros((L,), jnp.float32)
        return 0

    lax.fori_loop(0, NPAD // L, zbody, 0)

    thr = thr_v[pl.ds(0, L)][0]
    ones = jnp.ones((L,), jnp.float32)

    def body(i, off):
        ew = ew_v[pl.ds(i * L, L)]
        m = ew > thr
        r = row_v[pl.ds(i * L, L)]
        cc = col_v[pl.ds(i * L, L)]
        plsc.store_compressed(rowc_v.at[pl.ds(off, L)], r, mask=m)
        plsc.store_compressed(colc_v.at[pl.ds(off, L)], cc, mask=m)
        plsc.addupdate_scatter(deg_v, [cc], ones, mask=m)
        return off + jnp.sum(m.astype(jnp.int32))

    off = lax.fori_loop(0, EPT // L, body, jnp.int32(0))

    sent = jnp.full((L,), N, jnp.int32)
    for k in range(KCH // L):
        rowc_v[pl.ds(off + k * L, L)] = sent
        colc_v[pl.ds(off + k * L, L)] = sent

    lane = lax.iota(jnp.int32, L)
    cnt_v[...] = jnp.where(lane == 0, off, 0)

    pltpu.sync_copy(rowc_v, rowc_hbm.at[wid])
    pltpu.sync_copy(colc_v, colc_hbm.at[wid])
    pltpu.sync_copy(cnt_v, cnt_hbm.at[wid])
    pltpu.sync_copy(deg_v, degp_hbm.at[wid])


def _run_b2(ew, row, col, thr_arr):
    mesh = plsc.VectorSubcoreMesh(core_axis_name="c", subcore_axis_name="s")
    f = pl.kernel(
        _b2_body,
        out_type=[
            jax.ShapeDtypeStruct((NW, CAP), jnp.int32),
            jax.ShapeDtypeStruct((NW, CAP), jnp.int32),
            jax.ShapeDtypeStruct((NW, L), jnp.int32),
            jax.ShapeDtypeStruct((NW, NPAD), jnp.float32),
        ],
        mesh=mesh,
        compiler_params=pltpu.CompilerParams(needs_layout_passes=False),
        scratch_types=[
            pltpu.VMEM((EPT,), jnp.float32),
            pltpu.VMEM((EPT,), jnp.int32),
            pltpu.VMEM((EPT,), jnp.int32),
            pltpu.VMEM((CAP,), jnp.int32),
            pltpu.VMEM((CAP,), jnp.int32),
            pltpu.VMEM((NPAD,), jnp.float32),
            pltpu.VMEM((L,), jnp.float32),
            pltpu.VMEM((L,), jnp.int32),
        ],
    )
    return f(ew, row, col, thr_arr)


def _e0_body(h_ref, degp_ref, w_ref, xw_ref, xws_ref, dis_ref):
    i = pl.program_id(0)
    degsum = lax.dot_general(
        degp_ref[...], jnp.ones((NW, 1), jnp.float32),
        (((0,), (0,)), ((), ())), preferred_element_type=jnp.float32)
    dis = lax.rsqrt(degsum + 1.0)
    rid = i * BR + lax.broadcasted_iota(jnp.int32, (BR, 1), 0)
    dis = jnp.where(rid < N, dis, 0.0)
    xw = jnp.dot(h_ref[...], w_ref[...], preferred_element_type=jnp.float32)
    xw_ref[...] = xw
    xws_ref[...] = xw * dis
    dis_ref[...] = dis


def _run_e0(h_pad, degp, W_c1):
    grid = NPAD // BR
    return pl.pallas_call(
        _e0_body,
        grid=(grid,),
        in_specs=[
            pl.BlockSpec((BR, H), lambda i: (i, 0)),
            pl.BlockSpec((NW, BR), lambda i: (0, i)),
            pl.BlockSpec((H, H), lambda i: (0, 0)),
        ],
        out_specs=[
            pl.BlockSpec((BR, H), lambda i: (i, 0)),
            pl.BlockSpec((BR, H), lambda i: (i, 0)),
            pl.BlockSpec((BR, 1), lambda i: (i, 0)),
        ],
        out_shape=[
            jax.ShapeDtypeStruct((NPAD, H), jnp.float32),
            jax.ShapeDtypeStruct((NPAD, H), jnp.float32),
            jax.ShapeDtypeStruct((NPAD, 1), jnp.float32),
        ],
    )(h_pad, degp, W_c1)


def _d_body(xws_hbm, rowc_hbm, colc_hbm, cnt_hbm, s_hbm,
            agg_l, rid_v, cid_v, cntall_v, pr_v, pc_v, rows_v, sem, gsem):
    c = lax.axis_index("c")
    s = lax.axis_index("s")
    wid = s * NC + c
    lo = wid * SEG

    def zb(i, _):
        for k in range(H // L):
            agg_l[i, pl.ds(k * L, L)] = jnp.zeros((L,), jnp.float32)
        return 0

    lax.fori_loop(0, SEG, zb, 0)

    pltpu.sync_copy(cnt_hbm, cntall_v)

    sentr = jnp.full((L,), N, jnp.int32)
    sentc = jnp.zeros((L,), jnp.int32)

    def drain(off):
        for k in range(KCH // L):
            pr_v[pl.ds(off + k * L, L)] = sentr
            pc_v[pl.ds(off + k * L, L)] = sentc
        nchd = (off + (KCH - 1)) // KCH

        @pl.when(nchd > 0)
        def _():
            pltpu.async_copy(
                xws_hbm.at[pr_v.at[pl.ds(0, KCH)]], rows_v.at[0], gsem)

        def chunk(j, _):
            pp = j % 2
            pltpu.make_async_copy(
                xws_hbm.at[pr_v.at[pl.ds(j * KCH, KCH)]],
                rows_v.at[pp], gsem).wait()

            @pl.when(j + 1 < nchd)
            def _():
                pltpu.async_copy(
                    xws_hbm.at[pr_v.at[pl.ds((j + 1) * KCH, KCH)]],
                    rows_v.at[1 - pp], gsem)

            def acc_group(g, _):
                cv = pc_v[pl.ds(j * KCH + g * L, L)]
                for lane in range(L):
                    cl = cv[lane]
                    for k in range(H // L):
                        plsc.addupdate(
                            agg_l.at[cl, pl.ds(k * L, L)],
                            rows_v[pp, g * L + lane, pl.ds(k * L, L)])
                return 0

            lax.fori_loop(0, KCH // L, acc_group, 0)
            return 0

        lax.fori_loop(0, nchd, chunk, 0)
        return jnp.int32(0)

    def keep(off):
        return off

    def scan_subrows(buf_p, mq, off0):
        def q_loop(q, off1):
            for u in range(KCH // L):
                r16 = rid_v[buf_p, q, pl.ds(u * L, L)]
                c16 = cid_v[buf_p, q, pl.ds(u * L, L)]
                mloc = (c16 >= lo) & (c16 < lo + SEG)
                plsc.store_compressed(
                    pr_v.at[pl.ds(off1, L)], r16, mask=mloc)
                plsc.store_compressed(
                    pc_v.at[pl.ds(off1, L)], c16 - lo, mask=mloc)
                off1 = off1 + jnp.sum(mloc.astype(jnp.int32))
            return lax.cond(off1 >= PCAP, drain, keep, off1)

        return lax.fori_loop(0, mq, q_loop, off0)

    pltpu.async_copy(rowc_hbm.at[0, pl.ds(0, GR2)], rid_v.at[0], sem)
    pltpu.async_copy(colc_hbm.at[0, pl.ds(0, GR2)], cid_v.at[0], sem)

    def list_loop(w2, off):
        p2 = w2 % 2
        pltpu.make_async_copy(
            rowc_hbm.at[w2, pl.ds(0, GR2)], rid_v.at[p2], sem).wait()
        pltpu.make_async_copy(
            colc_hbm.at[w2, pl.ds(0, GR2)], cid_v.at[p2], sem).wait()

        @pl.when(w2 + 1 < NW)
        def _():
            pltpu.async_copy(
                rowc_hbm.at[w2 + 1, pl.ds(0, GR2)], rid_v.at[1 - p2], sem)
            pltpu.async_copy(
                colc_hbm.at[w2 + 1, pl.ds(0, GR2)], cid_v.at[1 - p2], sem)

        cnt2 = cntall_v[pl.ds(w2 * L, L)][0]
        nsub = (cnt2 + (KCH - 1)) // KCH
        off = scan_subrows(p2, jnp.minimum(nsub, GR2), off)

        ngr = (nsub + (GR2 - 1)) // GR2

        def ov_loop(g2, off2):
            pltpu.sync_copy(rowc_hbm.at[w2, pl.ds(g2 * GR2, GR2)],
                            rid_v.at[p2])
            pltpu.sync_copy(colc_hbm.at[w2, pl.ds(g2 * GR2, GR2)],
                            cid_v.at[p2])
            return scan_subrows(p2, jnp.minimum(nsub - g2 * GR2, GR2), off2)

        off = lax.fori_loop(1, ngr, ov_loop, off)
        return off

    off = lax.fori_loop(0, NW, list_loop, jnp.int32(0))
    off = drain(off)

    pltpu.sync_copy(agg_l, s_hbm.at[pl.ds(lo, SEG)])


def _run_d(xws, rowc3, colc3, cnt_flat):
    mesh = plsc.VectorSubcoreMesh(core_axis_name="c", subcore_axis_name="s")
    f = pl.kernel(
        _d_body,
        out_type=jax.ShapeDtypeStruct((NPAD, H), jnp.float32),
        mesh=mesh,
        compiler_params=pltpu.CompilerParams(needs_layout_passes=False),
        scratch_types=[
            pltpu.VMEM((SEG, H), jnp.float32),
            pltpu.VMEM((2, GR2, KCH), jnp.int32),
            pltpu.VMEM((2, GR2, KCH), jnp.int32),
            pltpu.VMEM((NW * L,), jnp.int32),
            pltpu.VMEM((PCAP + 2 * KCH,), jnp.int32),
            pltpu.VMEM((PCAP + 2 * KCH,), jnp.int32),
            pltpu.VMEM((2, KCH, H), jnp.float32),
            pltpu.SemaphoreType.DMA,
            pltpu.SemaphoreType.DMA,
        ],
    )
    return f(xws, rowc3, colc3, cnt_flat)


def _e1_body(s_ref, xw_ref, dis_ref, b_ref, w2_ref, h1_ref, xw2_ref, xws2_ref):
    ssum = s_ref[...]
    dis = dis_ref[...]
    h1 = jax.nn.relu(dis * ssum + xw_ref[...] * (dis * dis) + b_ref[...])
    xw2 = jnp.dot(h1, w2_ref[...], preferred_element_type=jnp.float32)
    h1_ref[...] = h1
    xw2_ref[...] = xw2
    xws2_ref[...] = xw2 * dis


def _run_e1(s1, xw1, dis, b_c1, W_c2):
    grid = NPAD // BR
    return pl.pallas_call(
        _e1_body,
        grid=(grid,),
        in_specs=[
            pl.BlockSpec((BR, H), lambda i: (i, 0)),
            pl.BlockSpec((BR, H), lambda i: (i, 0)),
            pl.BlockSpec((BR, 1), lambda i: (i, 0)),
            pl.BlockSpec((1, H), lambda i: (0, 0)),
            pl.BlockSpec((H, H), lambda i: (0, 0)),
        ],
        out_specs=[
            pl.BlockSpec((BR, H), lambda i: (i, 0)),
            pl.BlockSpec((BR, H), lambda i: (i, 0)),
            pl.BlockSpec((BR, H), lambda i: (i, 0)),
        ],
        out_shape=[
            jax.ShapeDtypeStruct((NPAD, H), jnp.float32),
            jax.ShapeDtypeStruct((NPAD, H), jnp.float32),
            jax.ShapeDtypeStruct((NPAD, H), jnp.float32),
        ],
    )(s1, xw1, dis, b_c1.reshape(1, H), W_c2)


def _f_body(s_ref, xw2_ref, dis_ref, bc2_ref, h_ref, h1_ref,
            wo1_ref, wo2_ref, wo3_ref, bo_ref, out_ref):
    ssum = s_ref[...]
    dis = dis_ref[...]
    h2 = jax.nn.relu(dis * ssum + xw2_ref[...] * (dis * dis) + bc2_ref[...])
    out = (jnp.dot(h_ref[...], wo1_ref[...], preferred_element_type=jnp.float32)
           + jnp.dot(h1_ref[...], wo2_ref[...], preferred_element_type=jnp.float32)
           + jnp.dot(h2, wo3_ref[...], preferred_element_type=jnp.float32)
           + bo_ref[...])
    out_ref[...] = out


def _run_f(s2, xw2, dis, b_c2, h_pad, h1, W_out, b_out):
    grid = NPAD // BR
    return pl.pallas_call(
        _f_body,
        grid=(grid,),
        in_specs=[
            pl.BlockSpec((BR, H), lambda i: (i, 0)),
            pl.BlockSpec((BR, H), lambda i: (i, 0)),
            pl.BlockSpec((BR, 1), lambda i: (i, 0)),
            pl.BlockSpec((1, H), lambda i: (0, 0)),
            pl.BlockSpec((BR, H), lambda i: (i, 0)),
            pl.BlockSpec((BR, H), lambda i: (i, 0)),
            pl.BlockSpec((H, 1), lambda i: (0, 0)),
            pl.BlockSpec((H, 1), lambda i: (0, 0)),
            pl.BlockSpec((H, 1), lambda i: (0, 0)),
            pl.BlockSpec((1, 1), lambda i: (0, 0)),
        ],
        out_specs=pl.BlockSpec((BR, 1), lambda i: (i, 0)),
        out_shape=jax.ShapeDtypeStruct((NPAD, 1), jnp.float32),
    )(s2, xw2, dis, b_c2.reshape(1, H), h_pad, h1,
      W_out[0:H], W_out[H:2 * H], W_out[2 * H:3 * H], b_out.reshape(1, 1))


def kernel(x, edge_index, W_in, b_in, W_imp1, b_imp1, W_imp2, b_imp2,
           W_c1, b_c1, W_c2, b_c2, W_out, b_out):
    row = edge_index[0]
    col = edge_index[1]
    x_pad = jnp.pad(x, ((0, NPAD - N), (0, 0)))

    h_pad, imp = _run_a(x_pad, W_in, b_in, W_imp1, b_imp1, W_imp2, b_imp2)
    imp_flat = imp.reshape(NPAD)

    ew, part = _run_b1(imp_flat, row, col)

    ssum = jnp.sum(part[:, 0])
    qsum = jnp.sum(part[:, 1])
    mean = ssum / E
    var = jnp.maximum(qsum - ssum * mean, 0.0) / (E - 1)
    thr = mean + jnp.sqrt(var)
    thr_arr = jnp.full((L,), thr, jnp.float32)

    rowc, colc, cnt, degp = _run_b2(ew, row, col, thr_arr)
    rowc3 = rowc.reshape(NW, CAP // KCH, KCH)
    colc3 = colc.reshape(NW, CAP // KCH, KCH)
    cnt_flat = cnt.reshape(NW * L)

    xw1, xws1, dis = _run_e0(h_pad, degp, W_c1)
    s1 = _run_d(xws1, rowc3, colc3, cnt_flat)
    h1, xw2, xws2 = _run_e1(s1, xw1, dis, b_c1, W_c2)
    s2 = _run_d(xws2, rowc3, colc3, cnt_flat)
    out = _run_f(s2, xw2, dis, b_c2, h_pad, h1, W_out, b_out)
    return out[:N, 0]
